# Initial kernel scaffold; baseline (speedup 1.0000x reference)
import jax
import jax.numpy as jnp
from jax.experimental import pallas as pl


def kernel(docs, emb_weight, lin_weight, lin_bias):
    raise NotImplementedError("write your pallas kernel here")


# speedup vs baseline: 96.0790x; 96.0790x over previous
import functools

import jax
import jax.numpy as jnp
from jax import lax
from jax.experimental import pallas as pl
from jax.experimental.pallas import tpu as pltpu
from jax.experimental.pallas import tpu_sc as plsc

_NC = 2
_NS = 16
_LN = 16
_NW = _NC * _NS


def _fold_body(emb_ref, lin_ref, m_ref):
    m_ref[...] = lax.dot_general(
        emb_ref[...], lin_ref[...], (((1,), (1,)), ((), ())),
        preferred_element_type=jnp.float32)


def _fold_table(emb, lin):
    return pl.pallas_call(
        _fold_body,
        out_shape=jax.ShapeDtypeStruct((emb.shape[0], lin.shape[0]), jnp.float32),
    )(emb, lin)


def _make_sc_bow(batch, seq_len, vocab, nclass):
    docs_per_w = batch // _NW
    groups = docs_per_w // _LN

    @functools.partial(
        pl.kernel,
        out_type=jax.ShapeDtypeStruct((batch * nclass,), jnp.float32),
        mesh=plsc.VectorSubcoreMesh(core_axis_name="c", subcore_axis_name="s"),
        compiler_params=pltpu.CompilerParams(needs_layout_passes=False),
        scratch_types=[
            pltpu.VMEM((vocab * nclass,), jnp.float32),
            pltpu.VMEM((_LN * seq_len,), jnp.int32),
            pltpu.VMEM((docs_per_w * nclass,), jnp.float32),
        ],
    )
    def sc_bow(docs_hbm, m_hbm, out_hbm, m_v, docs_v, out_v):
        wid = lax.axis_index("s") * _NC + lax.axis_index("c")
        base = wid * docs_per_w
        pltpu.sync_copy(m_hbm, m_v)
        iota = lax.iota(jnp.int32, _LN)
        row_off = iota * seq_len

        def g_body(g, carry):
            pltpu.sync_copy(
                docs_hbm.at[pl.ds((base + g * _LN) * seq_len, _LN * seq_len)],
                docs_v)

            def l_body(l, pacc):
                toks = plsc.load_gather(docs_v, [row_off + l])
                tb = toks * nclass
                return tuple(
                    pacc[c] + plsc.load_gather(m_v, [tb + c])
                    for c in range(nclass))

            pacc = lax.fori_loop(
                0, seq_len, l_body,
                tuple(jnp.zeros((_LN,), jnp.float32) for _ in range(nclass)))
            out_idx = (g * _LN + iota) * nclass
            for c in range(nclass):
                plsc.store_scatter(out_v, [out_idx + c], pacc[c])
            return carry

        lax.fori_loop(0, groups, g_body, 0)
        pltpu.sync_copy(
            out_v, out_hbm.at[pl.ds(base * nclass, docs_per_w * nclass)])

    return sc_bow


def _lsm_body(inv_len, s_ref, b_ref, o_ref):
    z = s_ref[...] * inv_len + b_ref[...]
    m = jnp.max(z, axis=0, keepdims=True)
    e = jnp.exp(z - m)
    lse = jnp.log(jnp.sum(e, axis=0, keepdims=True))
    o_ref[...] = z - m - lse


def kernel(docs, emb_weight, lin_weight, lin_bias):
    batch, seq_len = docs.shape
    vocab, _ = emb_weight.shape
    nclass = lin_weight.shape[0]

    m = _fold_table(emb_weight, lin_weight)
    sc_bow = _make_sc_bow(batch, seq_len, vocab, nclass)
    s_flat = sc_bow(docs.reshape(-1), m.reshape(-1))
    s = s_flat.reshape(batch, nclass)
    out = pl.pallas_call(
        functools.partial(_lsm_body, 1.0 / seq_len),
        out_shape=jax.ShapeDtypeStruct((batch, nclass), jnp.float32),
    )(s, lin_bias.reshape(1, nclass))
    return out


# speedup vs baseline: 117.9897x; 1.2280x over previous
import functools

import jax
import jax.numpy as jnp
from jax import lax
from jax.experimental import pallas as pl
from jax.experimental.pallas import tpu as pltpu
from jax.experimental.pallas import tpu_sc as plsc

_NC = 2
_NS = 16
_LN = 16
_NW = _NC * _NS

_UNROLL = 4


def _fold_body(emb_ref, lin_ref, m_ref):
    m_ref[...] = lax.dot_general(
        emb_ref[...], lin_ref[...], (((1,), (1,)), ((), ())),
        preferred_element_type=jnp.float32)


def _fold_table(emb, lin):
    return pl.pallas_call(
        _fold_body,
        out_shape=jax.ShapeDtypeStruct((emb.shape[0], lin.shape[0]), jnp.float32),
    )(emb, lin)


def _pack_pairs(m):
    u = lax.bitcast_convert_type(m.astype(jnp.bfloat16), jnp.uint16)
    u = u.astype(jnp.uint32)
    packed = u[:, 0::2] | (u[:, 1::2] << 16)
    return packed.astype(jnp.int32).reshape(-1)


def _make_sc_bow(batch, seq_len, vocab, nclass):
    npair = nclass // 2
    docs_per_w = batch // _NW
    groups = docs_per_w // _LN

    @functools.partial(
        pl.kernel,
        out_type=jax.ShapeDtypeStruct((batch, nclass), jnp.float32),
        mesh=plsc.VectorSubcoreMesh(core_axis_name="c", subcore_axis_name="s"),
        compiler_params=pltpu.CompilerParams(needs_layout_passes=False),
        scratch_types=[
            pltpu.VMEM((vocab * npair,), jnp.int32),
            pltpu.VMEM((_LN, seq_len), jnp.int32),
            pltpu.VMEM((_LN, seq_len), jnp.int32),
            pltpu.VMEM((docs_per_w, nclass), jnp.float32),
            pltpu.SemaphoreType.DMA,
            pltpu.SemaphoreType.DMA,
        ],
    )
    def sc_bow(docs_hbm, m_hbm, out_hbm, m_v, docs_v0, docs_v1, out_v,
               sem0, sem1):
        wid = lax.axis_index("s") * _NC + lax.axis_index("c")
        base = wid * docs_per_w
        pltpu.sync_copy(m_hbm, m_v)
        iota = lax.iota(jnp.int32, _LN)

        def docs_dma(g, buf_ref, sem):
            return pltpu.make_async_copy(
                docs_hbm.at[pl.ds(base + g * _LN, _LN), :], buf_ref, sem)

        def accumulate(g, buf_ref):
            def l_body(i, accs):
                new = list(accs)
                for j in range(_UNROLL):
                    l = i * _UNROLL + j
                    toks = plsc.load_gather(
                        buf_ref, [iota, jnp.full((_LN,), l, jnp.int32)])
                    tb = toks * npair
                    for k in range(npair):
                        w = plsc.load_gather(m_v, [tb + k])
                        new[k] = new[k] + plsc.bitcast(w, jnp.bfloat16)
                return tuple(new)

            accs = lax.fori_loop(
                0, seq_len // _UNROLL, l_body,
                tuple(jnp.zeros((2 * _LN,), jnp.bfloat16)
                      for _ in range(npair)))
            row = g * _LN + iota
            for k in range(npair):
                w = plsc.bitcast(accs[k], jnp.int32)
                f_even = plsc.bitcast(w << 16, jnp.float32)
                f_odd = plsc.bitcast(w & jnp.int32(-65536), jnp.float32)
                plsc.store_scatter(
                    out_v, [row, jnp.full((_LN,), 2 * k, jnp.int32)], f_even)
                plsc.store_scatter(
                    out_v, [row, jnp.full((_LN,), 2 * k + 1, jnp.int32)],
                    f_odd)

        docs_dma(0, docs_v0, sem0).start()

        def g2_body(h, carry):
            g0 = h * 2

            @pl.when(g0 + 1 < groups)
            def _():
                docs_dma(g0 + 1, docs_v1, sem1).start()

            docs_dma(g0, docs_v0, sem0).wait()
            accumulate(g0, docs_v0)

            @pl.when(g0 + 2 < groups)
            def _():
                docs_dma(g0 + 2, docs_v0, sem0).start()

            @pl.when(g0 + 1 < groups)
            def _():
                docs_dma(g0 + 1, docs_v1, sem1).wait()
                accumulate(g0 + 1, docs_v1)

            return carry

        lax.fori_loop(0, (groups + 1) // 2, g2_body, 0)
        pltpu.sync_copy(out_v, out_hbm.at[pl.ds(base, docs_per_w), :])

    return sc_bow


def _lsm_body(inv_len, s_ref, b_ref, o_ref):
    z = s_ref[...] * inv_len + b_ref[...]
    m = jnp.max(z, axis=0, keepdims=True)
    e = jnp.exp(z - m)
    lse = jnp.log(jnp.sum(e, axis=0, keepdims=True))
    o_ref[...] = z - m - lse


def kernel(docs, emb_weight, lin_weight, lin_bias):
    batch, seq_len = docs.shape
    vocab, _ = emb_weight.shape
    nclass = lin_weight.shape[0]

    m = _fold_table(emb_weight, lin_weight)
    m_packed = _pack_pairs(m)
    sc_bow = _make_sc_bow(batch, seq_len, vocab, nclass)
    s = sc_bow(docs, m_packed)
    out = pl.pallas_call(
        functools.partial(_lsm_body, 1.0 / seq_len),
        out_shape=jax.ShapeDtypeStruct((batch, nclass), jnp.float32),
    )(s, lin_bias.reshape(1, nclass))
    return out


# speedup vs baseline: 191.9357x; 1.6267x over previous
import functools

import jax
import jax.numpy as jnp
from jax import lax
from jax.experimental import pallas as pl
from jax.experimental.pallas import tpu as pltpu
from jax.experimental.pallas import tpu_sc as plsc

_NC = 2
_NS = 16
_LN = 16
_NW = _NC * _NS

_UNROLL = 4


def _fold_body(emb_ref, lin_ref, m_ref):
    m_ref[...] = lax.dot_general(
        emb_ref[...], lin_ref[...], (((1,), (1,)), ((), ())),
        preferred_element_type=jnp.float32)


def _fold_table(emb, lin):
    return pl.pallas_call(
        _fold_body,
        out_shape=jax.ShapeDtypeStruct((emb.shape[0], lin.shape[0]), jnp.float32),
    )(emb, lin)


def _pack_pairs(m):
    u = lax.bitcast_convert_type(m.astype(jnp.bfloat16), jnp.uint16)
    u = u.astype(jnp.uint32)
    packed = u[:, 0::2] | (u[:, 1::2] << 16)
    return packed.astype(jnp.int32).reshape(-1)


_SG = 128


def _make_sc_bow(batch, seq_len, vocab, nclass):
    npair = nclass // 2
    docs_per_w = batch // _NW
    sgroups = docs_per_w // _SG
    subs = _SG // _LN

    @functools.partial(
        pl.kernel,
        out_type=jax.ShapeDtypeStruct((batch, nclass), jnp.float32),
        mesh=plsc.VectorSubcoreMesh(core_axis_name="c", subcore_axis_name="s"),
        compiler_params=pltpu.CompilerParams(needs_layout_passes=False),
        scratch_types=[
            pltpu.VMEM((vocab * npair,), jnp.int32),
            pltpu.VMEM((seq_len, _SG), jnp.int32),
            pltpu.VMEM((seq_len, _SG), jnp.int32),
            pltpu.VMEM((docs_per_w, nclass), jnp.float32),
            pltpu.SemaphoreType.DMA,
            pltpu.SemaphoreType.DMA,
        ],
    )
    def sc_bow(docs_hbm, m_hbm, out_hbm, m_v, docs_v0, docs_v1, out_v,
               sem0, sem1):
        wid = lax.axis_index("s") * _NC + lax.axis_index("c")
        base = wid * docs_per_w
        pltpu.sync_copy(m_hbm, m_v)
        iota = lax.iota(jnp.int32, _LN)

        def docs_dma(sg, buf_ref, sem):
            return pltpu.make_async_copy(
                docs_hbm.at[:, pl.ds(base + sg * _SG, _SG)], buf_ref, sem)

        def accumulate(sg, buf_ref):
            for sub in range(subs):
                col = sub * _LN + iota

                def l_body(i, accs):
                    new = list(accs)
                    for j in range(_UNROLL):
                        l = i * _UNROLL + j
                        toks = plsc.load_gather(
                            buf_ref, [jnp.full((_LN,), l, jnp.int32), col])
                        tb = toks * npair
                        for k in range(npair):
                            w = plsc.load_gather(m_v, [tb + k])
                            new[k] = new[k] + plsc.bitcast(w, jnp.bfloat16)
                    return tuple(new)

                accs = lax.fori_loop(
                    0, seq_len // _UNROLL, l_body,
                    tuple(jnp.zeros((2 * _LN,), jnp.bfloat16)
                          for _ in range(npair)))
                row = sg * _SG + sub * _LN + iota
                for k in range(npair):
                    w = plsc.bitcast(accs[k], jnp.int32)
                    f_even = plsc.bitcast(w << 16, jnp.float32)
                    f_odd = plsc.bitcast(w & jnp.int32(-65536), jnp.float32)
                    plsc.store_scatter(
                        out_v, [row, jnp.full((_LN,), 2 * k, jnp.int32)],
                        f_even)
                    plsc.store_scatter(
                        out_v, [row, jnp.full((_LN,), 2 * k + 1, jnp.int32)],
                        f_odd)

        docs_dma(0, docs_v0, sem0).start()

        def g2_body(h, carry):
            g0 = h * 2

            @pl.when(g0 + 1 < sgroups)
            def _():
                docs_dma(g0 + 1, docs_v1, sem1).start()

            docs_dma(g0, docs_v0, sem0).wait()
            accumulate(g0, docs_v0)

            @pl.when(g0 + 2 < sgroups)
            def _():
                docs_dma(g0 + 2, docs_v0, sem0).start()

            @pl.when(g0 + 1 < sgroups)
            def _():
                docs_dma(g0 + 1, docs_v1, sem1).wait()
                accumulate(g0 + 1, docs_v1)

            return carry

        lax.fori_loop(0, (sgroups + 1) // 2, g2_body, 0)
        pltpu.sync_copy(out_v, out_hbm.at[pl.ds(base, docs_per_w), :])

    return sc_bow


def _lsm_body(inv_len, s_ref, b_ref, o_ref):
    z = s_ref[...] * inv_len + b_ref[...]
    m = jnp.max(z, axis=0, keepdims=True)
    e = jnp.exp(z - m)
    lse = jnp.log(jnp.sum(e, axis=0, keepdims=True))
    o_ref[...] = z - m - lse


def kernel(docs, emb_weight, lin_weight, lin_bias):
    batch, seq_len = docs.shape
    vocab, _ = emb_weight.shape
    nclass = lin_weight.shape[0]

    m = _fold_table(emb_weight, lin_weight)
    m_packed = _pack_pairs(m)
    docs_t = docs.T
    sc_bow = _make_sc_bow(batch, seq_len, vocab, nclass)
    s = sc_bow(docs_t, m_packed)
    out = pl.pallas_call(
        functools.partial(_lsm_body, 1.0 / seq_len),
        out_shape=jax.ShapeDtypeStruct((batch, nclass), jnp.float32),
    )(s, lin_bias.reshape(1, nclass))
    return out
